# baseline (device time: 40560 ns/iter reference)
import jax
import jax.numpy as jnp
from jax import lax
from jax.experimental import pallas as pl
from jax.experimental.pallas import tpu as pltpu

T = 1024
D = 1024
V_LOCAL = 8192
HALF = T // 2

CS = [16, 32, 64, 96, 96, 96, 64, 32, 16]
OFFS = [sum(CS[:k]) for k in range(len(CS))]
K = len(CS)
assert sum(CS) == HALF


def kernel(ids, E):
    my_y = lax.axis_index("y")

    local = ids - my_y * V_LOCAL
    mask = (local >= 0) & (local < V_LOCAL)
    safe = jnp.clip(local, 0, V_LOCAL - 1).astype(jnp.int32)
    maskf = mask.astype(jnp.float32)[:, None]

    def body(safe_ref, maskf_ref, e_ref, out_ref, *scratch):
        g_refs = scratch[:K]
        comm_ref = scratch[K]
        g_sems, y_send, y_recv, x_send, x_recv = scratch[K + 1:]
        my_x = lax.axis_index("x")
        my_y = lax.axis_index("y")
        y_nbr = (my_x, 1 - my_y)
        x_nbr = (1 - my_x, my_y)
        my_base = my_x * HALF

        barrier_sem = pltpu.get_barrier_semaphore()
        for nbr in (y_nbr, x_nbr):
            pl.semaphore_signal(
                barrier_sem, inc=1, device_id=nbr,
                device_id_type=pl.DeviceIdType.MESH,
            )
        pl.semaphore_wait(barrier_sem, 2)

        def issue_gather(k):
            base = OFFS[k]

            def issue_row(i, _):
                pltpu.make_async_copy(
                    e_ref.at[pl.ds(safe_ref[my_base + base + i], 1), :],
                    g_refs[k].at[pl.ds(i, 1), :],
                    g_sems.at[k],
                ).start()
                return 0

            lax.fori_loop(0, CS[k], issue_row, 0, unroll=8)

        def wait_gather(k):
            pltpu.make_async_copy(
                e_ref.at[pl.ds(0, CS[k]), :], g_refs[k], g_sems.at[k]
            ).wait()

        y_rdmas = []
        x_rdmas = []

        def process(k):
            sl = pl.ds(OFFS[k], CS[k])
            msl = pl.ds(my_base + OFFS[k], CS[k])
            out_sl = pl.ds(my_base + OFFS[k], CS[k])
            y_rdmas[k].wait_recv()
            out_ref[out_sl, :] = jnp.where(
                maskf_ref[msl, :] != 0.0, g_refs[k][:, :], comm_ref[sl, :]
            )
            r = pltpu.make_async_remote_copy(
                src_ref=out_ref.at[out_sl, :],
                dst_ref=out_ref.at[out_sl, :],
                send_sem=x_send.at[k],
                recv_sem=x_recv.at[k],
                device_id=x_nbr,
                device_id_type=pl.DeviceIdType.MESH,
            )
            r.start()
            x_rdmas.append(r)

        issue_gather(0)
        for k in range(K):
            if k + 1 < K:
                issue_gather(k + 1)
            sl = pl.ds(OFFS[k], CS[k])
            wait_gather(k)
            r = pltpu.make_async_remote_copy(
                src_ref=g_refs[k],
                dst_ref=comm_ref.at[sl, :],
                send_sem=y_send.at[k],
                recv_sem=y_recv.at[k],
                device_id=y_nbr,
                device_id_type=pl.DeviceIdType.MESH,
            )
            r.start()
            y_rdmas.append(r)
            if k >= 1:
                process(k - 1)
        process(K - 1)

        for k in range(K):
            y_rdmas[k].wait_send()
            x_rdmas[k].wait_send()
            x_rdmas[k].wait_recv()

    return pl.pallas_call(
        body,
        out_shape=jax.ShapeDtypeStruct((T, D), jnp.float32),
        in_specs=[
            pl.BlockSpec(memory_space=pltpu.SMEM),
            pl.BlockSpec(memory_space=pltpu.VMEM),
            pl.BlockSpec(memory_space=pl.ANY),
        ],
        out_specs=pl.BlockSpec(memory_space=pltpu.VMEM),
        scratch_shapes=(
            [pltpu.VMEM((CS[k], D), jnp.float32) for k in range(K)]
            + [pltpu.VMEM((HALF, D), jnp.float32)]
            + [pltpu.SemaphoreType.DMA((K,))] * 5
        ),
        compiler_params=pltpu.CompilerParams(collective_id=0),
    )(safe, maskf, E)


# device time: 38125 ns/iter; 1.0639x vs baseline; 1.0639x over previous
import jax
import jax.numpy as jnp
from jax import lax
from jax.experimental import pallas as pl
from jax.experimental.pallas import tpu as pltpu

T = 1024
D = 1024
V_LOCAL = 8192
HALF = T // 2

CS = [16, 48] + [64] * 6 + [48, 16]
OFFS = [sum(CS[:k]) for k in range(len(CS))]
K = len(CS)
assert sum(CS) == HALF


def kernel(ids, E):
    my_y = lax.axis_index("y")

    local = ids - my_y * V_LOCAL
    mask = (local >= 0) & (local < V_LOCAL)
    safe = jnp.clip(local, 0, V_LOCAL - 1).astype(jnp.int32)
    maskf = mask.astype(jnp.float32)[:, None]

    def body(safe_ref, maskf_ref, e_ref, out_ref, *scratch):
        g_refs = scratch[:K]
        comm_ref = scratch[K]
        g_sems, y_send, y_recv, x_send, x_recv = scratch[K + 1:]
        my_x = lax.axis_index("x")
        my_y = lax.axis_index("y")
        y_nbr = (my_x, 1 - my_y)
        x_nbr = (1 - my_x, my_y)
        my_base = my_x * HALF

        barrier_sem = pltpu.get_barrier_semaphore()
        for nbr in (y_nbr, x_nbr):
            pl.semaphore_signal(
                barrier_sem, inc=1, device_id=nbr,
                device_id_type=pl.DeviceIdType.MESH,
            )
        pl.semaphore_wait(barrier_sem, 2)

        def issue_gather(k):
            base = OFFS[k]

            def issue_row(i, _):
                pltpu.make_async_copy(
                    e_ref.at[pl.ds(safe_ref[my_base + base + i], 1), :],
                    g_refs[k].at[pl.ds(i, 1), :],
                    g_sems.at[k],
                ).start()
                return 0

            lax.fori_loop(0, CS[k], issue_row, 0, unroll=8)

        def wait_gather(k):
            pltpu.make_async_copy(
                e_ref.at[pl.ds(0, CS[k]), :], g_refs[k], g_sems.at[k]
            ).wait()

        y_rdmas = []
        x_rdmas = []

        def process(k):
            sl = pl.ds(OFFS[k], CS[k])
            msl = pl.ds(my_base + OFFS[k], CS[k])
            out_sl = pl.ds(my_base + OFFS[k], CS[k])
            y_rdmas[k].wait_recv()
            out_ref[out_sl, :] = jnp.where(
                maskf_ref[msl, :] != 0.0, g_refs[k][:, :], comm_ref[sl, :]
            )
            r = pltpu.make_async_remote_copy(
                src_ref=out_ref.at[out_sl, :],
                dst_ref=out_ref.at[out_sl, :],
                send_sem=x_send.at[k],
                recv_sem=x_recv.at[k],
                device_id=x_nbr,
                device_id_type=pl.DeviceIdType.MESH,
            )
            r.start()
            x_rdmas.append(r)

        issue_gather(0)
        for k in range(K):
            if k + 1 < K:
                issue_gather(k + 1)
            sl = pl.ds(OFFS[k], CS[k])
            wait_gather(k)
            r = pltpu.make_async_remote_copy(
                src_ref=g_refs[k],
                dst_ref=comm_ref.at[sl, :],
                send_sem=y_send.at[k],
                recv_sem=y_recv.at[k],
                device_id=y_nbr,
                device_id_type=pl.DeviceIdType.MESH,
            )
            r.start()
            y_rdmas.append(r)
            if k >= 1:
                process(k - 1)
        process(K - 1)

        for k in range(K):
            y_rdmas[k].wait_send()
            x_rdmas[k].wait_send()
            x_rdmas[k].wait_recv()

    return pl.pallas_call(
        body,
        out_shape=jax.ShapeDtypeStruct((T, D), jnp.float32),
        in_specs=[
            pl.BlockSpec(memory_space=pltpu.SMEM),
            pl.BlockSpec(memory_space=pltpu.VMEM),
            pl.BlockSpec(memory_space=pl.ANY),
        ],
        out_specs=pl.BlockSpec(memory_space=pltpu.VMEM),
        scratch_shapes=(
            [pltpu.VMEM((CS[k], D), jnp.float32) for k in range(K)]
            + [pltpu.VMEM((HALF, D), jnp.float32)]
            + [pltpu.SemaphoreType.DMA((K,))] * 5
        ),
        compiler_params=pltpu.CompilerParams(collective_id=0),
    )(safe, maskf, E)


# device time: 37244 ns/iter; 1.0890x vs baseline; 1.0237x over previous
import jax
import jax.numpy as jnp
from jax import lax
from jax.experimental import pallas as pl
from jax.experimental.pallas import tpu as pltpu

T = 1024
D = 1024
V_LOCAL = 8192
HALF = T // 2

CS = [16, 48] + [64] * 6 + [48, 16]
OFFS = [sum(CS[:k]) for k in range(len(CS))]
K = len(CS)
assert sum(CS) == HALF


def kernel(ids, E):
    def body(ids_ref, e_ref, out_ref, *scratch):
        g_refs = scratch[:K]
        comm_ref = scratch[K]
        g_sems, y_send, y_recv, x_send, x_recv = scratch[K + 1:]
        my_x = lax.axis_index("x")
        my_y = lax.axis_index("y")
        y_nbr = (my_x, 1 - my_y)
        x_nbr = (1 - my_x, my_y)
        my_base = my_x * HALF
        off = my_y * V_LOCAL

        barrier_sem = pltpu.get_barrier_semaphore()
        for nbr in (y_nbr, x_nbr):
            pl.semaphore_signal(
                barrier_sem, inc=1, device_id=nbr,
                device_id_type=pl.DeviceIdType.MESH,
            )
        pl.semaphore_wait(barrier_sem, 2)

        def issue_gather(k):
            base = OFFS[k]
            g_refs[k][...] = jnp.zeros_like(g_refs[k])

            def issue_row(i, _):
                rid = ids_ref[my_base + base + i] - off
                valid = jnp.logical_and(rid >= 0, rid < V_LOCAL)
                src_row = lax.max(0, lax.min(rid, V_LOCAL - 1))
                dst_row = lax.select(valid, i, CS[k])
                pltpu.make_async_copy(
                    e_ref.at[pl.ds(src_row, 1), :],
                    g_refs[k].at[pl.ds(dst_row, 1), :],
                    g_sems.at[k],
                ).start()
                return 0

            lax.fori_loop(0, CS[k], issue_row, 0, unroll=8)

        def wait_gather(k):
            pltpu.make_async_copy(
                e_ref.at[pl.ds(0, CS[k]), :],
                g_refs[k].at[pl.ds(0, CS[k]), :],
                g_sems.at[k],
            ).wait()

        y_rdmas = []
        x_rdmas = []

        def process(k):
            sl = pl.ds(OFFS[k], CS[k])
            out_sl = pl.ds(my_base + OFFS[k], CS[k])
            y_rdmas[k].wait_recv()
            out_ref[out_sl, :] = (
                g_refs[k][pl.ds(0, CS[k]), :] + comm_ref[sl, :]
            )
            r = pltpu.make_async_remote_copy(
                src_ref=out_ref.at[out_sl, :],
                dst_ref=out_ref.at[out_sl, :],
                send_sem=x_send.at[k],
                recv_sem=x_recv.at[k],
                device_id=x_nbr,
                device_id_type=pl.DeviceIdType.MESH,
            )
            r.start()
            x_rdmas.append(r)

        issue_gather(0)
        for k in range(K):
            if k + 1 < K:
                issue_gather(k + 1)
            wait_gather(k)
            r = pltpu.make_async_remote_copy(
                src_ref=g_refs[k].at[pl.ds(0, CS[k]), :],
                dst_ref=comm_ref.at[pl.ds(OFFS[k], CS[k]), :],
                send_sem=y_send.at[k],
                recv_sem=y_recv.at[k],
                device_id=y_nbr,
                device_id_type=pl.DeviceIdType.MESH,
            )
            r.start()
            y_rdmas.append(r)
            if k >= 1:
                process(k - 1)
        process(K - 1)

        for k in range(K):
            y_rdmas[k].wait_send()
            x_rdmas[k].wait_send()
            x_rdmas[k].wait_recv()

    return pl.pallas_call(
        body,
        out_shape=jax.ShapeDtypeStruct((T, D), jnp.float32),
        in_specs=[
            pl.BlockSpec(memory_space=pltpu.SMEM),
            pl.BlockSpec(memory_space=pl.ANY),
        ],
        out_specs=pl.BlockSpec(memory_space=pltpu.VMEM),
        scratch_shapes=(
            [pltpu.VMEM((CS[k] + 1, D), jnp.float32) for k in range(K)]
            + [pltpu.VMEM((HALF, D), jnp.float32)]
            + [pltpu.SemaphoreType.DMA((K,))] * 5
        ),
        compiler_params=pltpu.CompilerParams(collective_id=0),
    )(ids, E)
